# Initial kernel scaffold; baseline (speedup 1.0000x reference)
import jax
import jax.numpy as jnp
from jax.experimental import pallas as pl


def kernel(x, params):
    raise NotImplementedError("write your pallas kernel here")


# speedup vs baseline: 4.8499x; 4.8499x over previous
import functools

import jax
import jax.numpy as jnp
from jax import lax
from jax.experimental import pallas as pl
from jax.experimental.pallas import tpu as pltpu

N = 1024
KNN = 20
RADIUS = 0.1
F32 = jnp.float32
NEG_BIG = -9e15
_INF = float('inf')
_MINF = float('-inf')


def _relu(a):
    return jnp.maximum(a, jnp.float32(0.0))


def _dot(a, b):
    return jnp.dot(a, b, preferred_element_type=F32)


def _dot_nt(a, b, precision=None):
    return lax.dot_general(a, b, (((1,), (1,)), ((), ())),
                           preferred_element_type=F32, precision=precision)


def _dot_exact(a, b):
    return jnp.dot(a, b, preferred_element_type=F32,
                   precision=lax.Precision.HIGHEST)


def _pdist_sq(f):
    gram = _dot_nt(f, f)
    t = f * f
    w = t.shape[1]
    while w > 1:
        h = w // 2
        r = w - h
        if r > h:
            t = jnp.concatenate([t[:, :h] + t[:, r:w], t[:, h:r]], axis=1)
        else:
            t = t[:, :h] + t[:, h:w]
        w = r
    return t - 2.0 * gram + t.T


def _argmin_onehot(key_act, iota_f):
    thr = jnp.min(key_act, axis=1, keepdims=True)
    is_min = key_act <= thr
    first = jnp.min(jnp.where(is_min, iota_f, jnp.float32(3e38)),
                    axis=1, keepdims=True)
    return jnp.where(is_min & (iota_f == first), jnp.float32(1.0),
                     jnp.float32(0.0))


def _topk_mask(key, sel_ref, k=KNN):
    sel_ref[...] = jnp.zeros(sel_ref.shape, F32)
    iota_f = lax.broadcasted_iota(jnp.int32, key.shape, 1).astype(F32)

    def body(i, c):
        key_act = jnp.where(sel_ref[...] > 0, _INF, key)
        sel_ref[...] = sel_ref[...] + _argmin_onehot(key_act, iota_f)
        return c

    lax.fori_loop(0, k, body, 0, unroll=False)
    return sel_ref[...]


def _knn_edge_max(key, f, w1t, sel_ref, m_ref, k=KNN):
    sel_ref[...] = jnp.zeros(sel_ref.shape, F32)
    m_ref[...] = jnp.full(m_ref.shape, _MINF, F32)
    iota_f = lax.broadcasted_iota(jnp.int32, key.shape, 1).astype(F32)

    def body(i, c):
        key_act = jnp.where(sel_ref[...] > 0, _INF, key)
        new = _argmin_onehot(key_act, iota_f)
        sel_ref[...] = sel_ref[...] + new
        nbr = _dot_exact(new, f)
        feat = jnp.concatenate([nbr - f, f], axis=1)
        m_ref[...] = jnp.maximum(m_ref[...], _dot(feat, w1t))
        return c

    lax.fori_loop(0, k, body, 0, unroll=False)
    return m_ref[...]


def _softmax_rows(e):
    e = e - jnp.max(e, axis=1, keepdims=True)
    ex = jnp.exp(e)
    return ex / jnp.sum(ex, axis=1, keepdims=True)


def _stn_body(xt_ref, w1, b1, w2, b2, w3, b3, fw1, fb1, fw2, fb2, fw3, fb3,
              out_ref):
    xt = xt_ref[0]
    h = _relu(_dot(xt, w1[...]) + b1[...])
    h = _relu(_dot(h, w2[...]) + b2[...])
    h = _relu(_dot(h, w3[...]) + b3[...])
    g = jnp.max(h, axis=0, keepdims=True)
    g = _relu(_dot(g, fw1[...]) + fb1[...])
    g = _relu(_dot(g, fw2[...]) + fb2[...])
    t = _dot(g, fw3[...]) + fb3[...]
    lane = lax.broadcasted_iota(jnp.int32, (1, 9), 1)
    t = t + jnp.where(lane % 4 == 0, jnp.float32(1.0), jnp.float32(0.0))
    xtb = xt.astype(jnp.bfloat16).astype(F32)
    tb = t.astype(jnp.bfloat16).astype(F32)
    cols = []
    for cix in range(3):
        col = jnp.zeros((N, 1), F32)
        for dix in range(3):
            s = jnp.sum(jnp.where(lane == 3 * cix + dix, tb,
                                  jnp.float32(0.0)),
                        axis=1, keepdims=True)
            col = col + xtb[:, dix:dix + 1] * s
        cols.append(col)
    out_ref[0] = jnp.concatenate(cols + [xt[:, 3:]], axis=1)


def _edgeconv_body(ft_ref, wt, out_ref, sel_ref, m_ref):
    f = ft_ref[0]
    dsqc = jnp.maximum(_pdist_sq(f), 1e-12)
    d = jnp.sqrt(dsqc)
    cnt = jnp.sum(jnp.where(d <= RADIUS, jnp.float32(1.0),
                            jnp.float32(0.0)), axis=1, keepdims=True)
    key = jnp.where(cnt >= float(KNN), d, dsqc)
    m = _knn_edge_max(key, f, wt[...], sel_ref, m_ref)
    out_ref[0] = _relu(m)


def _gat_body(f1_ref, f2_ref, f3_ref, xyz_ref, fuw, fub, wcat, asrc, adst,
              pw, pb, out_ref, sel_ref):
    fcat = jnp.concatenate([f1_ref[0], f2_ref[0], f3_ref[0]], axis=1)
    h0 = _relu(_dot(fcat, fuw[...]) + fub[...])

    xyz = xyz_ref[0]
    dsq = _pdist_sq(xyz)
    mf = _topk_mask(dsq, sel_ref)
    adj = jnp.maximum(mf, mf.T) > 0.0

    wh_all = _dot(h0, wcat[...])
    heads = []
    for hix in range(4):
        wh = wh_all[:, hix * 64:(hix + 1) * 64]
        s1 = _dot(wh, asrc[...][:, hix:hix + 1])
        s2t = _dot_nt(adst[...][hix:hix + 1, :], wh)
        e = s1 + s2t
        e = jnp.where(e > 0, e, jnp.float32(0.2) * e)
        e = jnp.where(adj, e, NEG_BIG)
        att = _softmax_rows(e)
        o = _dot(att, wh)
        heads.append(jnp.where(o > 0, o, jnp.exp(o) - 1.0))
    gat = _dot(jnp.concatenate(heads, axis=1), pw[...]) + pb[...]
    out_ref[0] = h0 + gat


def _ln_rows(a, g, b, eps=1e-5):
    mu = jnp.mean(a, axis=1, keepdims=True)
    var = jnp.mean((a - mu) ** 2, axis=1, keepdims=True)
    return (a - mu) / jnp.sqrt(var + eps) * g + b


def _xfmr_body(h_ref, wq, bq, wk, bk, wv, bv, wo, bo, ln1g, ln1b,
               f1, fb1, f2, fb2, ln2g, ln2b, out_ref):
    h = h_ref[0]
    q = _dot(h, wq[...]) + bq[...]
    k = _dot(h, wk[...]) + bk[...]
    v = _dot(h, wv[...]) + bv[...]
    heads = []
    for hix in range(4):
        sl = slice(hix * 64, (hix + 1) * 64)
        att = _softmax_rows(_dot_nt(q[:, sl], k[:, sl]) * jnp.float32(0.125))
        heads.append(_dot(att, v[:, sl]))
    o = _dot(jnp.concatenate(heads, axis=1), wo[...]) + bo[...]
    h = _ln_rows(h + o, ln1g[...], ln1b[...])
    ff = _dot(_relu(_dot(h, f1[...]) + fb1[...]), f2[...]) + fb2[...]
    out_ref[0] = _ln_rows(h + ff, ln2g[...], ln2b[...])


def _head_body(h_ref, cw1, cb1, cw2, cb2, out_ref):
    x = h_ref[...]
    gmax = jnp.max(x, axis=1)
    z = jnp.mean(x, axis=1, keepdims=True)

    def body(_, z):
        diff = x - z
        dist = jnp.sqrt(jnp.sum(diff * diff, axis=2, keepdims=True))
        dist = jnp.maximum(dist, jnp.float32(1e-5))
        w = 1.0 / dist
        w = w / jnp.sum(w, axis=1, keepdims=True)
        return jnp.sum(w * x, axis=1, keepdims=True)

    z = lax.fori_loop(0, 100, body, z, unroll=False)
    g = jnp.concatenate([gmax, z[:, 0, :]], axis=1)
    g = _relu(_dot(g, cw1[...]) + cb1[...])
    out_ref[...] = _dot(g, cw2[...]) + cb2[...]


def _bspec(shape):
    nd = len(shape)
    return pl.BlockSpec((1,) + shape[1:],
                        lambda b: (b,) + (0,) * (nd - 1))


def _wspec(shape):
    nd = len(shape)
    return pl.BlockSpec(shape, lambda b: (0,) * nd)


def _batch_call(body, ins, wts, out_shape, batch=8, scratch=()):
    return pl.pallas_call(
        body,
        grid=(batch,),
        in_specs=[_bspec(a.shape) for a in ins] + [_wspec(w.shape)
                                                   for w in wts],
        out_specs=_bspec(out_shape),
        out_shape=jax.ShapeDtypeStruct((batch,) + out_shape[1:], F32),
        scratch_shapes=list(scratch),
        compiler_params=pltpu.CompilerParams(
            dimension_semantics=("arbitrary",)),
    )(*ins, *wts)


def kernel(x, params):
    p = params
    B = x.shape[0]
    xt = jnp.transpose(x, (0, 2, 1))

    def r1(v):
        return v.reshape(1, -1)

    stn_wts = [p['s_w1'].T, r1(p['s_b1']), p['s_w2'].T, r1(p['s_b2']),
               p['s_w3'].T, r1(p['s_b3']), p['s_fw1'].T, r1(p['s_fb1']),
               p['s_fw2'].T, r1(p['s_fb2']), p['s_fw3'].T, r1(p['s_fb3'])]
    x0t = _batch_call(_stn_body, [xt], stn_wts, (B, N, 10), batch=B)

    def edge(ft, w, o_dim):
        scratch = (pltpu.VMEM((N, N), F32), pltpu.VMEM((N, o_dim), F32))
        return _batch_call(_edgeconv_body, [ft], [w.T],
                           (B, N, o_dim), batch=B, scratch=scratch)

    f1 = edge(x0t, p['e_w1'], 64)
    f2 = edge(f1, p['e_w2'], 64)
    f3 = edge(f2, p['e_w3'], 128)

    xyzt = x0t[:, :, :3]
    wcat = jnp.concatenate([p['g_W%d' % i] for i in range(4)], axis=1)
    asrc = jnp.concatenate([p['g_a%d' % i][:64, :] for i in range(4)],
                           axis=1)
    adst = jnp.concatenate([p['g_a%d' % i][64:, :].T for i in range(4)],
                           axis=0)
    gat_wts = [p['fu_w'].T, r1(p['fu_b']), wcat, asrc, adst,
               p['g_pw'].T, r1(p['g_pb'])]
    h = _batch_call(_gat_body, [f1, f2, f3, xyzt], gat_wts, (B, N, 256),
                    batch=B, scratch=(pltpu.VMEM((N, N), F32),))

    xf_wts = [p['t_wq'].T, r1(p['t_bq']), p['t_wk'].T, r1(p['t_bk']),
              p['t_wv'].T, r1(p['t_bv']), p['t_wo'].T, r1(p['t_bo']),
              r1(p['ln1_g']), r1(p['ln1_b']), p['t_f1'].T, r1(p['t_fb1']),
              p['t_f2'].T, r1(p['t_fb2']), r1(p['ln2_g']), r1(p['ln2_b'])]
    h = _batch_call(_xfmr_body, [h], xf_wts, (B, N, 256), batch=B)

    head_wts = [p['c_w1'].T, r1(p['c_b1']), p['c_w2'].T, r1(p['c_b2'])]
    out = pl.pallas_call(
        _head_body,
        out_shape=jax.ShapeDtypeStruct((B, 4), F32),
    )(h, *head_wts)
    return out
